# baseline (speedup 1.0000x reference)
import jax
import jax.numpy as jnp
from jax import lax
from jax.experimental import pallas as pl
from jax.experimental.pallas import tpu as pltpu

N_E = 256
E_DIM = 1024
N_ROWS = 16384
B = 8
P = N_ROWS // B
BETA = 0.25


PH = 1024
SH = 2048


def _vq_body(sm_ref, emb_ref, w_ref, b_ref, zc_ref, zch_ref,
             out_ref, oh_ref, idx_ref, loss_ref, perp_ref,
             acc_ref, cnt_ref):
    h = pl.program_id(1)
    t = pl.program_id(0) * 2 + h
    n_steps = 2 * B
    std = sm_ref[0, 0]
    mean = sm_ref[0, 1]
    ew = emb_ref[:] * std + mean
    ewn = jnp.sum(ew * ew, axis=1)

    zb = zc_ref[0]
    zft = jnp.transpose(zb).reshape(PH, E_DIM)
    rn = jnp.sum(zft * zft, axis=1)
    g = lax.dot_general(zft, ew, (((1,), (1,)), ((), ())),
                        preferred_element_type=jnp.float32)
    d = (rn[:, None] + ewn[None, :]) - 2.0 * g

    iota1 = lax.broadcasted_iota(jnp.int32, (PH, N_E), 1)
    minv = jnp.min(d, axis=1, keepdims=True)
    idx = jnp.min(jnp.where(d == minv, iota1, N_E), axis=1)
    idx_ref[...] = idx.reshape(1, 1, PH)

    oh = (iota1 == idx[:, None]).astype(jnp.float32)
    oh_ref[:] = oh

    zq = lax.dot_general(oh, ew, (((1,), (0,)), ((), ())),
                         preferred_element_type=jnp.float32)
    zch = zch_ref[0]
    zqc = zq.reshape(256, 4096)
    diff = zqc - zch
    tile_s = jnp.sum(diff * diff)
    zqste = zch + diff

    part = lax.dot_general(w_ref[:], zqste, (((1,), (0,)), ((), ())),
                           preferred_element_type=jnp.float32)

    @pl.when(h == 0)
    def _():
        out_ref[0] = part

    @pl.when(h == 1)
    def _():
        out_ref[0] = (out_ref[0] + part) + b_ref[:]

    cnt = jnp.sum(oh, axis=0).reshape(1, N_E)
    prev_s = jnp.where(t == 0, 0.0, acc_ref[0, 0])
    acc_ref[0, 0] = prev_s + tile_s
    prev_c = jnp.where(t == 0, jnp.zeros((1, N_E), jnp.float32), cnt_ref[...])
    cnt_ref[...] = prev_c + cnt

    @pl.when(t == n_steps - 1)
    def _():
        m = acc_ref[0, 0] / float(N_ROWS * E_DIM)
        loss_ref[0, 0] = m + BETA * m
        e_mean = cnt_ref[...] / float(N_ROWS)
        perp_ref[0, 0] = jnp.exp(-jnp.sum(e_mean * jnp.log(e_mean + 1e-10)))


def kernel(z, embedding_weight, mean_param, std_param, conv_w, conv_b, noise):
    std = jnp.abs(std_param) + noise
    mean = jnp.mean(mean_param)
    sm = jnp.stack([std, mean]).reshape(1, 2).astype(jnp.float32)

    zc = z.reshape(B, 512, 4096)
    w2 = conv_w[:, :, 0, 0]
    bias2 = conv_b[:, None]

    out, oh, idx3, loss, perp = pl.pallas_call(
        _vq_body,
        grid=(B, 2),
        in_specs=[
            pl.BlockSpec(memory_space=pltpu.SMEM),
            pl.BlockSpec((N_E, E_DIM), lambda t, h: (0, 0)),
            pl.BlockSpec((512, 256), lambda t, h: (0, h)),
            pl.BlockSpec((512, 1), lambda t, h: (0, 0)),
            pl.BlockSpec((1, 512, SH), lambda t, h: (t, 0, h)),
            pl.BlockSpec((1, 256, 4096), lambda t, h: (t, h, 0)),
        ],
        out_specs=[
            pl.BlockSpec((1, 512, 4096), lambda t, h: (t, 0, 0)),
            pl.BlockSpec((PH, N_E), lambda t, h: (2 * t + h, 0)),
            pl.BlockSpec((1, 1, PH), lambda t, h: (2 * t + h, 0, 0)),
            pl.BlockSpec(memory_space=pltpu.SMEM),
            pl.BlockSpec(memory_space=pltpu.SMEM),
        ],
        out_shape=[
            jax.ShapeDtypeStruct((B, 512, 4096), jnp.float32),
            jax.ShapeDtypeStruct((N_ROWS, N_E), jnp.float32),
            jax.ShapeDtypeStruct((2 * B, 1, PH), jnp.int32),
            jax.ShapeDtypeStruct((1, 1), jnp.float32),
            jax.ShapeDtypeStruct((1, 1), jnp.float32),
        ],
        scratch_shapes=[
            pltpu.SMEM((1, 1), jnp.float32),
            pltpu.VMEM((1, N_E), jnp.float32),
        ],
        compiler_params=pltpu.CompilerParams(
            vmem_limit_bytes=100 * 1024 * 1024,
        ),
    )(sm, embedding_weight, w2, bias2, zc, zc)

    z_q = out.reshape(z.shape)
    min_encoding_indices = idx3.reshape(N_ROWS, 1)
    return (z_q, loss.reshape(()), (perp.reshape(()), oh, min_encoding_indices))


# speedup vs baseline: 1.0974x; 1.0974x over previous
import jax
import jax.numpy as jnp
from jax import lax
from jax.experimental import pallas as pl
from jax.experimental.pallas import tpu as pltpu

N_E = 256
E_DIM = 1024
N_ROWS = 16384
B = 8
P = N_ROWS // B
BETA = 0.25


def _vq_body(sm_ref, emb_ref, w_ref, b_ref, zc_ref,
             out_ref, oh_ref, idx_ref, loss_ref, perp_ref,
             acc_ref, cnt_ref):
    t = pl.program_id(0)
    std = sm_ref[0, 0]
    mean = sm_ref[0, 1]
    ew = emb_ref[:] * std + mean
    ewn = jnp.sum(ew * ew, axis=1)

    zb = zc_ref[0]
    zft = jnp.transpose(zb).reshape(P, E_DIM)
    rn = jnp.sum(zft * zft, axis=1)
    g = lax.dot_general(zft, ew, (((1,), (1,)), ((), ())),
                        preferred_element_type=jnp.float32)
    d = (rn[:, None] + ewn[None, :]) - 2.0 * g

    iota1 = lax.broadcasted_iota(jnp.int32, (P, N_E), 1)
    minv = jnp.min(d, axis=1, keepdims=True)
    idx = jnp.min(jnp.where(d == minv, iota1, N_E), axis=1)
    idx_ref[...] = idx.reshape(1, 1, P)

    oh = (iota1 == idx[:, None]).astype(jnp.float32)
    oh_ref[:] = oh

    zq = lax.dot_general(oh, ew, (((1,), (0,)), ((), ())),
                         preferred_element_type=jnp.float32)
    zr = zb.reshape(P, E_DIM)
    diff = zq - zr
    tile_s = jnp.sum(diff * diff)
    zqste = zr + diff

    zqb = zqste.reshape(512, 4096)
    o = lax.dot_general(w_ref[:], zqb, (((1,), (0,)), ((), ())),
                        preferred_element_type=jnp.float32)
    out_ref[0] = o + b_ref[:]

    cnt = jnp.sum(oh, axis=0).reshape(1, N_E)
    prev_s = jnp.where(t == 0, 0.0, acc_ref[0, 0])
    acc_ref[0, 0] = prev_s + tile_s
    prev_c = jnp.where(t == 0, jnp.zeros((1, N_E), jnp.float32), cnt_ref[...])
    cnt_ref[...] = prev_c + cnt

    @pl.when(t == B - 1)
    def _():
        m = acc_ref[0, 0] / float(N_ROWS * E_DIM)
        loss_ref[0, 0] = m + BETA * m
        e_mean = cnt_ref[...] / float(N_ROWS)
        perp_ref[0, 0] = jnp.exp(-jnp.sum(e_mean * jnp.log(e_mean + 1e-10)))


def kernel(z, embedding_weight, mean_param, std_param, conv_w, conv_b, noise):
    std = jnp.abs(std_param) + noise
    mean = jnp.mean(mean_param)
    sm = jnp.stack([std, mean]).reshape(1, 2).astype(jnp.float32)

    zc = z.reshape(B, 512, 4096)
    w2 = conv_w[:, :, 0, 0]
    bias2 = conv_b[:, None]

    out, oh, idx3, loss, perp = pl.pallas_call(
        _vq_body,
        grid=(B,),
        in_specs=[
            pl.BlockSpec(memory_space=pltpu.SMEM),
            pl.BlockSpec((N_E, E_DIM), lambda t: (0, 0)),
            pl.BlockSpec((512, 512), lambda t: (0, 0)),
            pl.BlockSpec((512, 1), lambda t: (0, 0)),
            pl.BlockSpec((1, 512, 4096), lambda t: (t, 0, 0)),
        ],
        out_specs=[
            pl.BlockSpec((1, 512, 4096), lambda t: (t, 0, 0)),
            pl.BlockSpec((P, N_E), lambda t: (t, 0)),
            pl.BlockSpec((1, 1, P), lambda t: (t, 0, 0)),
            pl.BlockSpec(memory_space=pltpu.SMEM),
            pl.BlockSpec(memory_space=pltpu.SMEM),
        ],
        out_shape=[
            jax.ShapeDtypeStruct((B, 512, 4096), jnp.float32),
            jax.ShapeDtypeStruct((N_ROWS, N_E), jnp.float32),
            jax.ShapeDtypeStruct((B, 1, P), jnp.int32),
            jax.ShapeDtypeStruct((1, 1), jnp.float32),
            jax.ShapeDtypeStruct((1, 1), jnp.float32),
        ],
        scratch_shapes=[
            pltpu.SMEM((1, 1), jnp.float32),
            pltpu.VMEM((1, N_E), jnp.float32),
        ],
        compiler_params=pltpu.CompilerParams(
            vmem_limit_bytes=100 * 1024 * 1024,
        ),
    )(sm, embedding_weight, w2, bias2, zc)

    z_q = out.reshape(z.shape)
    min_encoding_indices = idx3.reshape(N_ROWS, 1)
    return (z_q, loss.reshape(()), (perp.reshape(()), oh, min_encoding_indices))
